# baseline (speedup 1.0000x reference)
import functools

import jax
import jax.numpy as jnp
from jax import lax
from jax.experimental import pallas as pl
from jax.experimental.pallas import tpu as pltpu
from jax.experimental.pallas import tpu_sc as plsc

NC = 2
NS = 16
NW = NC * NS
LANES = 16


def _sc_gather_sum(table, idx3):
    nw, nchunk, ch = idx3.shape
    bpw = nchunk // 2
    v, d = table.shape
    nd = d // LANES

    def body(table_hbm, idx_hbm, out_hbm, idx_v, buf_v, out_v, sem):
        wid = lax.axis_index("s") * NC + lax.axis_index("c")
        pltpu.sync_copy(idx_hbm.at[wid], idx_v)

        def row_body(r, carry):
            cp0 = pltpu.async_copy(
                table_hbm.at[idx_v.at[2 * r]], buf_v.at[pl.ds(0, ch)], sem)
            cp1 = pltpu.async_copy(
                table_hbm.at[idx_v.at[2 * r + 1]], buf_v.at[pl.ds(ch, ch)], sem)
            cp0.wait()
            cp1.wait()

            def acc_body(i, accs):
                return tuple(
                    accs[c]
                    + buf_v[i, pl.ds(LANES * c, LANES)]
                    + buf_v[ch + i, pl.ds(LANES * c, LANES)]
                    for c in range(nd))

            accs = lax.fori_loop(
                0, ch, acc_body,
                tuple(jnp.zeros((LANES,), jnp.float32) for _ in range(nd)))
            for c in range(nd):
                out_v[r, pl.ds(LANES * c, LANES)] = accs[c]
            return carry

        lax.fori_loop(0, bpw, row_body, 0)
        pltpu.sync_copy(out_v, out_hbm.at[pl.ds(wid * bpw, bpw)])

    run = pl.kernel(
        body,
        out_type=jax.ShapeDtypeStruct((nw * bpw, d), jnp.float32),
        mesh=plsc.VectorSubcoreMesh(core_axis_name="c", subcore_axis_name="s"),
        scratch_types=[
            pltpu.VMEM((nchunk, ch), jnp.int32),
            pltpu.VMEM((2 * ch, d), jnp.float32),
            pltpu.VMEM((bpw, d), jnp.float32),
            pltpu.SemaphoreType.DMA,
        ],
        compiler_params=pltpu.CompilerParams(use_tc_tiling_on_sc=False),
    )
    return run(table, idx3)


def _mlp(sums, w1, b1, w2t, b2, inv_l):
    b, d = sums.shape
    h = w1.shape[1]

    def body(s_ref, w1_ref, b1_ref, w2_ref, b2_ref, o_ref):
        feats = s_ref[...] * inv_l
        hid = jnp.dot(feats, w1_ref[...], preferred_element_type=jnp.float32)
        hid = jnp.maximum(hid + b1_ref[...], 0.0)
        logits = (jnp.sum(hid * w2_ref[...], axis=1, keepdims=True)
                  + b2_ref[...])
        mx = jnp.max(logits, axis=1, keepdims=True)
        e = jnp.exp(logits - mx)
        o_ref[...] = e / jnp.sum(e, axis=1, keepdims=True)

    return pl.pallas_call(
        body,
        out_shape=jax.ShapeDtypeStruct((b, 1), jnp.float32),
    )(sums, w1, b1, w2t, b2)


def kernel(inputs, table, W1, b1, W2, b2):
    b, l = inputs.shape
    idx3 = inputs.astype(jnp.int32).reshape(NW, (b // NW) * 2, l // 2)
    sums = _sc_gather_sum(table, idx3)
    return _mlp(sums, W1, b1.reshape(1, -1), W2.reshape(1, -1),
                b2.reshape(1, 1), 1.0 / l)


# speedup vs baseline: 1.2009x; 1.2009x over previous
import functools

import jax
import jax.numpy as jnp
from jax import lax
from jax.experimental import pallas as pl
from jax.experimental.pallas import tpu as pltpu
from jax.experimental.pallas import tpu_sc as plsc

NC = 2
NS = 16
NW = NC * NS
LANES = 16


def _sc_gather_sum(table, idx3):
    nw, nchunk, ch = idx3.shape
    bpw = nchunk // 2
    v, d = table.shape
    nd = d // LANES

    nbuf = 4
    unroll = 4

    def body(table_hbm, idx_hbm, out_hbm, idx_v, buf_v, out_v, *sems):
        wid = lax.axis_index("s") * NC + lax.axis_index("c")
        pltpu.sync_copy(idx_hbm.at[wid], idx_v)

        def fire(r, s):
            pltpu.async_copy(
                table_hbm.at[idx_v.at[2 * r]],
                buf_v.at[s, pl.ds(0, ch)], sems[s])
            pltpu.async_copy(
                table_hbm.at[idx_v.at[2 * r + 1]],
                buf_v.at[s, pl.ds(ch, ch)], sems[s])

        def drain(s):
            pltpu.make_async_copy(
                table_hbm.at[pl.ds(0, 2 * ch)], buf_v.at[s], sems[s]).wait()

        for s in range(nbuf):
            fire(s, s)

        def group_body(g, carry):
            for s in range(nbuf):
                r = g * nbuf + s
                drain(s)

                def acc_body(i, accs):
                    new = list(accs)
                    for u in range(unroll):
                        row = i * unroll + u
                        for c in range(nd):
                            new[c] = (new[c]
                                      + buf_v[s, row, pl.ds(LANES * c, LANES)]
                                      + buf_v[s, ch + row, pl.ds(LANES * c, LANES)])
                    return tuple(new)

                accs = lax.fori_loop(
                    0, ch // unroll, acc_body,
                    tuple(jnp.zeros((LANES,), jnp.float32) for _ in range(nd)))
                for c in range(nd):
                    out_v[r, pl.ds(LANES * c, LANES)] = accs[c]

                @pl.when(r + nbuf < bpw)
                def _():
                    fire(r + nbuf, s)
            return carry

        lax.fori_loop(0, bpw // nbuf, group_body, 0)
        pltpu.sync_copy(out_v, out_hbm.at[pl.ds(wid * bpw, bpw)])

    run = pl.kernel(
        body,
        out_type=jax.ShapeDtypeStruct((nw * bpw, d), jnp.float32),
        mesh=plsc.VectorSubcoreMesh(core_axis_name="c", subcore_axis_name="s"),
        scratch_types=[
            pltpu.VMEM((nchunk, ch), jnp.int32),
            pltpu.VMEM((nbuf, 2 * ch, d), jnp.float32),
            pltpu.VMEM((bpw, d), jnp.float32),
        ] + [pltpu.SemaphoreType.DMA] * nbuf,
        compiler_params=pltpu.CompilerParams(use_tc_tiling_on_sc=False),
    )
    return run(table, idx3)


def _mlp(sums, w1, b1, w2t, b2, inv_l):
    b, d = sums.shape
    h = w1.shape[1]

    def body(s_ref, w1_ref, b1_ref, w2_ref, b2_ref, o_ref):
        feats = s_ref[...] * inv_l
        hid = jnp.dot(feats, w1_ref[...], preferred_element_type=jnp.float32)
        hid = jnp.maximum(hid + b1_ref[...], 0.0)
        logits = (jnp.sum(hid * w2_ref[...], axis=1, keepdims=True)
                  + b2_ref[...])
        mx = jnp.max(logits, axis=1, keepdims=True)
        e = jnp.exp(logits - mx)
        o_ref[...] = e / jnp.sum(e, axis=1, keepdims=True)

    return pl.pallas_call(
        body,
        out_shape=jax.ShapeDtypeStruct((b, 1), jnp.float32),
    )(sums, w1, b1, w2t, b2)


def kernel(inputs, table, W1, b1, W2, b2):
    b, l = inputs.shape
    idx3 = inputs.astype(jnp.int32).reshape(NW, (b // NW) * 2, l // 2)
    sums = _sc_gather_sum(table, idx3)
    return _mlp(sums, W1, b1.reshape(1, -1), W2.reshape(1, -1),
                b2.reshape(1, 1), 1.0 / l)
